# Initial kernel scaffold; baseline (speedup 1.0000x reference)
import jax
import jax.numpy as jnp
from jax.experimental import pallas as pl


def kernel(src, spatial_shapes, level_start_index, pos, topk_inds, params):
    raise NotImplementedError("write your pallas kernel here")


# speedup vs baseline: 55.2828x; 55.2828x over previous
import functools
import numpy as np
import jax
import jax.numpy as jnp
from jax import lax
from jax.experimental import pallas as pl
from jax.experimental.pallas import tpu as pltpu
from jax.experimental.pallas import tpu_sc as plsc

D_MODEL = 256
N_HEADS = 8
HEAD_DIM = 32
N_LEVELS = 4
N_POINTS = 4
D_FFN = 1024
SPATIAL = [(64, 64), (32, 32), (16, 16), (8, 8)]
B = 2
K_TOP = 1024
N_TOTAL = sum(h * w for h, w in SPATIAL)

NW = 32
QPW = (B * K_TOP) // NW
ROWS_PW = N_TOTAL // 16

def _col_consts():
    j = lax.broadcasted_iota(jnp.int32, (1, 128), 1)
    hj = jnp.right_shift(j, 4)
    lj = jnp.bitwise_and(jnp.right_shift(j, 2), 3)
    shift = 6 - lj
    wci = jnp.left_shift(jnp.int32(1), shift)
    wcf = wci.astype(jnp.float32)
    startc = jnp.where(lj == 0, 0, jnp.where(lj == 1, 4096,
                       jnp.where(lj == 2, 5120, 5376)))
    return hj, wci, wcf, startc


def _head_onehot():
    r = jnp.right_shift(lax.broadcasted_iota(jnp.int32, (128, 8), 0), 4)
    c = lax.broadcasted_iota(jnp.int32, (128, 8), 1)
    return (r == c).astype(jnp.float32)


def _value_body(src_ref, w1_ref, b1_ref, w2_ref, b2_ref, v1_ref, v2_ref):
    x = src_ref[...]
    v1_ref[...] = jnp.dot(x, w1_ref[...], preferred_element_type=jnp.float32) + b1_ref[...]
    v2_ref[...] = jnp.dot(x, w2_ref[...], preferred_element_type=jnp.float32) + b2_ref[...]


def _value_proj(src2d, w1, b1, w2, b2):
    blk = 680
    grid = (B * N_TOTAL) // blk
    return pl.pallas_call(
        _value_body,
        grid=(grid,),
        in_specs=[
            pl.BlockSpec((blk, D_MODEL), lambda i: (i, 0)),
            pl.BlockSpec((D_MODEL, D_MODEL), lambda i: (0, 0)),
            pl.BlockSpec((1, D_MODEL), lambda i: (0, 0)),
            pl.BlockSpec((D_MODEL, D_MODEL), lambda i: (0, 0)),
            pl.BlockSpec((1, D_MODEL), lambda i: (0, 0)),
        ],
        out_specs=[
            pl.BlockSpec((blk, D_MODEL), lambda i: (i, 0)),
            pl.BlockSpec((blk, D_MODEL), lambda i: (i, 0)),
        ],
        out_shape=[
            jax.ShapeDtypeStruct((B * N_TOTAL, D_MODEL), jnp.float32),
            jax.ShapeDtypeStruct((B * N_TOTAL, D_MODEL), jnp.float32),
        ],
    )(src2d, w1, b1.reshape(1, -1), w2, b2.reshape(1, -1))


def _qprep_body(tgt_ref, pos_ref, topk_ref, wox_ref, box_ref, woy_ref, boy_ref,
                wat_ref, bat_ref, slx_ref, sly_ref, att_ref, idx_ref, w_ref):
    blkq = tgt_ref.shape[0]
    pid = pl.program_id(0)
    b = (pid * blkq) // K_TOP
    query = tgt_ref[...] + pos_ref[...]

    offx = jnp.dot(query, wox_ref[...], preferred_element_type=jnp.float32) + box_ref[...]
    offy = jnp.dot(query, woy_ref[...], preferred_element_type=jnp.float32) + boy_ref[...]
    logits = jnp.dot(query, wat_ref[...], preferred_element_type=jnp.float32) + bat_ref[...]

    m = jnp.max(logits, axis=-1, keepdims=True)
    e = jnp.exp(logits - m)
    gh = _head_onehot()
    denom8 = jnp.dot(e, gh, preferred_element_type=jnp.float32)
    db = jnp.dot(denom8, gh.T, preferred_element_type=jnp.float32)
    att = e / db
    att_ref[...] = att

    n = topk_ref[...]
    lvl1 = (n >= 4096).astype(jnp.int32)
    lvl2 = (n >= 5120).astype(jnp.int32)
    lvl3 = (n >= 5376).astype(jnp.int32)
    start = jnp.where(n < 4096, 0, jnp.where(n < 5120, 4096,
                      jnp.where(n < 5376, 5120, 5376)))
    shift = 6 - (lvl1 + lvl2 + lvl3)
    wn = jnp.left_shift(jnp.int32(1), shift)
    rem = n - start
    row = jnp.right_shift(rem, shift)
    col = jnp.bitwise_and(rem, wn - 1)
    wn_f = wn.astype(jnp.float32)
    refx = (col.astype(jnp.float32) + 0.5) / wn_f
    refy = (row.astype(jnp.float32) + 0.5) / wn_f

    hj, wci, wcf, startc = _col_consts()
    hcf = wcf

    slx = refx + offx * (1.0 / wcf)
    sly = refy + offy * (1.0 / hcf)
    slx_ref[...] = slx
    sly_ref[...] = sly

    x = slx * wcf - 0.5
    y = sly * hcf - 0.5
    x0 = jnp.floor(x)
    y0 = jnp.floor(y)
    lwx = x - x0
    lwy = y - y0

    bn8 = b * (N_TOTAL * 8)
    corners = [
        (0.0, 0.0, (1 - lwx) * (1 - lwy)),
        (1.0, 0.0, lwx * (1 - lwy)),
        (0.0, 1.0, (1 - lwx) * lwy),
        (1.0, 1.0, lwx * lwy),
    ]
    for ci, (dx, dy, cw) in enumerate(corners):
        xi = x0 + dx
        yi = y0 + dy
        valid = ((xi >= 0) & (xi <= wcf - 1) & (yi >= 0) & (yi <= hcf - 1))
        xc = jnp.clip(xi, 0, wcf - 1).astype(jnp.int32)
        yc = jnp.clip(yi, 0, hcf - 1).astype(jnp.int32)
        loc = startc + yc * wci + xc
        idx = bn8 + loc * 8 + hj
        w = cw * att * valid.astype(jnp.float32)
        idx_ref[:, ci * 128:(ci + 1) * 128] = idx
        w_ref[:, ci * 128:(ci + 1) * 128] = w


def _qprep(tgt, pos_g, topk2d, wox, box, woy, boy, wat, bat):
    blkq = 256
    grid = (B * K_TOP) // blkq
    return pl.pallas_call(
        _qprep_body,
        grid=(grid,),
        in_specs=[
            pl.BlockSpec((blkq, D_MODEL), lambda i: (i, 0)),
            pl.BlockSpec((blkq, D_MODEL), lambda i: (i, 0)),
            pl.BlockSpec((blkq, 1), lambda i: (i, 0)),
            pl.BlockSpec((D_MODEL, 128), lambda i: (0, 0)),
            pl.BlockSpec((1, 128), lambda i: (0, 0)),
            pl.BlockSpec((D_MODEL, 128), lambda i: (0, 0)),
            pl.BlockSpec((1, 128), lambda i: (0, 0)),
            pl.BlockSpec((D_MODEL, 128), lambda i: (0, 0)),
            pl.BlockSpec((1, 128), lambda i: (0, 0)),
        ],
        out_specs=[
            pl.BlockSpec((blkq, 128), lambda i: (i, 0)),
            pl.BlockSpec((blkq, 128), lambda i: (i, 0)),
            pl.BlockSpec((blkq, 128), lambda i: (i, 0)),
            pl.BlockSpec((blkq, 512), lambda i: (i, 0)),
            pl.BlockSpec((blkq, 512), lambda i: (i, 0)),
        ],
        out_shape=[
            jax.ShapeDtypeStruct((B * K_TOP, 128), jnp.float32),
            jax.ShapeDtypeStruct((B * K_TOP, 128), jnp.float32),
            jax.ShapeDtypeStruct((B * K_TOP, 128), jnp.float32),
            jax.ShapeDtypeStruct((B * K_TOP, 512), jnp.int32),
            jax.ShapeDtypeStruct((B * K_TOP, 512), jnp.float32),
        ],
    )(tgt, pos_g, topk2d, wox, box.reshape(1, -1), woy, boy.reshape(1, -1),
      wat, bat.reshape(1, -1))


def _ln(x, g, b):
    m = jnp.mean(x, axis=-1, keepdims=True)
    xc = x - m
    v = jnp.mean(xc * xc, axis=-1, keepdims=True)
    return xc / jnp.sqrt(v + 1e-5) * g + b


def _post_body(attn_ref, tgt_ref, wo_ref, bo_ref, g1_ref, be1_ref,
               wf1_ref, bf1_ref, wf2_ref, bf2_ref, g2_ref, be2_ref,
               wvn_ref, bvn_ref, tgt_out_ref, vupd_ref):
    a = jnp.dot(attn_ref[...], wo_ref[...], preferred_element_type=jnp.float32) + bo_ref[...]
    t = _ln(tgt_ref[...] + a, g1_ref[...], be1_ref[...])
    ff1 = jnp.maximum(jnp.dot(t, wf1_ref[...], preferred_element_type=jnp.float32) + bf1_ref[...], 0.0)
    ff = jnp.dot(ff1, wf2_ref[...], preferred_element_type=jnp.float32) + bf2_ref[...]
    t2 = _ln(t + ff, g2_ref[...], be2_ref[...])
    tgt_out_ref[...] = t2
    vupd_ref[...] = jnp.dot(t2, wvn_ref[...], preferred_element_type=jnp.float32) + bvn_ref[...]


def _post(attn, tgt, wo, bo, g1, be1, wf1, bf1, wf2, bf2, g2, be2, wvn, bvn):
    blkq = 256
    grid = (B * K_TOP) // blkq
    full = lambda r, c: pl.BlockSpec((r, c), lambda i: (0, 0))
    return pl.pallas_call(
        _post_body,
        grid=(grid,),
        in_specs=[
            pl.BlockSpec((blkq, D_MODEL), lambda i: (i, 0)),
            pl.BlockSpec((blkq, D_MODEL), lambda i: (i, 0)),
            full(D_MODEL, D_MODEL), full(1, D_MODEL),
            full(1, D_MODEL), full(1, D_MODEL),
            full(D_MODEL, D_FFN), full(1, D_FFN),
            full(D_FFN, D_MODEL), full(1, D_MODEL),
            full(1, D_MODEL), full(1, D_MODEL),
            full(D_MODEL, D_MODEL), full(1, D_MODEL),
        ],
        out_specs=[
            pl.BlockSpec((blkq, D_MODEL), lambda i: (i, 0)),
            pl.BlockSpec((blkq, D_MODEL), lambda i: (i, 0)),
        ],
        out_shape=[
            jax.ShapeDtypeStruct((B * K_TOP, D_MODEL), jnp.float32),
            jax.ShapeDtypeStruct((B * K_TOP, D_MODEL), jnp.float32),
        ],
    )(attn, tgt, wo, bo.reshape(1, -1), g1.reshape(1, -1), be1.reshape(1, -1),
      wf1, bf1.reshape(1, -1), wf2, bf2.reshape(1, -1), g2.reshape(1, -1),
      be2.reshape(1, -1), wvn, bvn.reshape(1, -1))


def _sc_gather_body(src_hbm, pos_hbm, gidx_hbm, tgt_out, pos_out,
                    idx_v, rows_v, sem):
    c = lax.axis_index("c")
    s = lax.axis_index("s")
    wid = s * 2 + c
    base = wid * QPW
    pltpu.sync_copy(gidx_hbm.at[wid], idx_v)
    pltpu.async_copy(src_hbm.at[idx_v], rows_v, sem).wait()
    pltpu.sync_copy(rows_v, tgt_out.at[pl.ds(base, QPW)])
    pltpu.async_copy(pos_hbm.at[idx_v], rows_v, sem).wait()
    pltpu.sync_copy(rows_v, pos_out.at[pl.ds(base, QPW)])


def _sc_gather(src2d, pos2d, gidx):
    mesh = plsc.VectorSubcoreMesh(core_axis_name="c", subcore_axis_name="s")
    f = functools.partial(
        pl.kernel,
        out_type=[
            jax.ShapeDtypeStruct((B * K_TOP, D_MODEL), jnp.float32),
            jax.ShapeDtypeStruct((B * K_TOP, D_MODEL), jnp.float32),
        ],
        mesh=mesh,
        scratch_types=[
            pltpu.VMEM((QPW,), jnp.int32),
            pltpu.VMEM((QPW, D_MODEL), jnp.float32),
            pltpu.SemaphoreType.DMA,
        ],
    )(_sc_gather_body)
    return f(src2d, pos2d, gidx)


def _sc_attn_body(v_hbm, idx_hbm, w_hbm, out_hbm, idxw_v, w_v, rows_v, out_v, sem):
    c = lax.axis_index("c")
    s = lax.axis_index("s")
    wid = s * 2 + c
    base = wid * QPW
    pltpu.sync_copy(idx_hbm.at[pl.ds(base * 512, QPW * 512)], idxw_v)
    pltpu.sync_copy(w_hbm.at[pl.ds(base * 512, QPW * 512)], w_v)

    def body(q, carry):
        cps = []
        for c4 in range(4):
            cps.append(pltpu.async_copy(
                v_hbm.at[idxw_v.at[pl.ds(q * 512 + c4 * 128, 128)]],
                rows_v.at[pl.ds(c4 * 128, 128)], sem))
        for cp in cps:
            cp.wait()
        for h in range(8):
            acc0 = jnp.zeros((16,), jnp.float32)
            acc1 = jnp.zeros((16,), jnp.float32)
            for c4 in range(4):
                w16 = w_v[pl.ds(q * 512 + c4 * 128 + h * 16, 16)]
                for k in range(16):
                    p = c4 * 128 + h * 16 + k
                    wb = w16[k]
                    acc0 = acc0 + wb * rows_v[p, pl.ds(0, 16)]
                    acc1 = acc1 + wb * rows_v[p, pl.ds(16, 16)]
            out_v[q, pl.ds(h * 32, 16)] = acc0
            out_v[q, pl.ds(h * 32 + 16, 16)] = acc1
        return carry

    lax.fori_loop(0, QPW, body, 0)
    pltpu.sync_copy(out_v, out_hbm.at[pl.ds(base, QPW)])


def _sc_attn(v_rows, idx512, w512):
    mesh = plsc.VectorSubcoreMesh(core_axis_name="c", subcore_axis_name="s")
    f = functools.partial(
        pl.kernel,
        out_type=jax.ShapeDtypeStruct((B * K_TOP, D_MODEL), jnp.float32),
        mesh=mesh,
        scratch_types=[
            pltpu.VMEM((QPW * 512,), jnp.int32),
            pltpu.VMEM((QPW * 512,), jnp.float32),
            pltpu.VMEM((512, HEAD_DIM), jnp.float32),
            pltpu.VMEM((QPW, D_MODEL), jnp.float32),
            pltpu.SemaphoreType.DMA,
        ],
        compiler_params=pltpu.CompilerParams(use_tc_tiling_on_sc=False),
    )(_sc_attn_body)
    return f(v_rows, idx512.reshape(-1), w512.reshape(-1))


def _sc_merge_body(base_hbm, rows_hbm, gidx_hbm, out_hbm, buf_v, idx_v, rows_v, sem):
    c = lax.axis_index("c")
    s = lax.axis_index("s")
    start = c * N_TOTAL + s * 336
    pltpu.sync_copy(base_hbm.at[pl.ds(start, 336)], buf_v)
    pltpu.sync_copy(buf_v, out_hbm.at[pl.ds(start, 336)])

    @pl.when(s == 0)
    def _():
        tail = c * N_TOTAL + 16 * 336
        pltpu.sync_copy(base_hbm.at[pl.ds(tail, 64)], buf_v.at[pl.ds(0, 64)])
        pltpu.sync_copy(buf_v.at[pl.ds(0, 64)], out_hbm.at[pl.ds(tail, 64)])

    plsc.subcore_barrier()
    chunk = c * 16 + s
    rbase = chunk * QPW
    pltpu.sync_copy(gidx_hbm.at[chunk], idx_v)
    pltpu.sync_copy(rows_hbm.at[pl.ds(rbase, QPW)], rows_v)
    pltpu.async_copy(rows_v, out_hbm.at[idx_v], sem).wait()


def _sc_merge(base2d, rows2d, gidx):
    mesh = plsc.VectorSubcoreMesh(core_axis_name="c", subcore_axis_name="s")
    f = functools.partial(
        pl.kernel,
        out_type=jax.ShapeDtypeStruct((B * N_TOTAL, D_MODEL), jnp.float32),
        mesh=mesh,
        scratch_types=[
            pltpu.VMEM((336, D_MODEL), jnp.float32),
            pltpu.VMEM((QPW,), jnp.int32),
            pltpu.VMEM((QPW, D_MODEL), jnp.float32),
            pltpu.SemaphoreType.DMA,
        ],
    )(_sc_merge_body)
    return f(base2d, rows2d, gidx)


def kernel(src, spatial_shapes, level_start_index, pos, topk_inds, params):
    src2d = src.reshape(B * N_TOTAL, D_MODEL)
    pos2d = pos.reshape(B * N_TOTAL, D_MODEL)
    gidx = (topk_inds + jnp.arange(B, dtype=jnp.int32)[:, None] * N_TOTAL)
    gidx = gidx.reshape(NW, QPW)
    topk2d = topk_inds.reshape(B * K_TOP, 1)

    p0, p1 = params
    v1, v2 = _value_proj(src2d, p0['W_val'], p0['b_val'], p1['W_val'], p1['b_val'])
    tgt, pos_g = _sc_gather(src2d, pos2d, gidx)

    sls = []
    aws = []
    v_cur = v1
    for li, p in enumerate(params):
        wox = p['W_off'][:, 0::2]
        box = p['b_off'][0::2]
        woy = p['W_off'][:, 1::2]
        boy = p['b_off'][1::2]
        slx, sly, att, idx512, w512 = _qprep(
            tgt, pos_g, topk2d, wox, box, woy, boy, p['W_att'], p['b_att'])
        sls.append(jnp.stack([slx, sly], -1).reshape(B, K_TOP, N_HEADS, N_LEVELS, N_POINTS, 2))
        aws.append(att.reshape(B, K_TOP, N_HEADS, N_LEVELS, N_POINTS))

        attn = _sc_attn(v_cur.reshape(B * N_TOTAL * N_HEADS, HEAD_DIM), idx512, w512)

        pn = params[min(li + 1, len(params) - 1)]
        tgt, vupd = _post(attn, tgt, p['W_out'], p['b_out'], p['g1'], p['be1'],
                          p['W_ff1'], p['b_ff1'], p['W_ff2'], p['b_ff2'],
                          p['g2'], p['be2'], pn['W_val'], pn['b_val'])
        if li == 0:
            v_cur = _sc_merge(v2, vupd, gidx)

    out = _sc_merge(src2d, tgt, gidx).reshape(B, N_TOTAL, D_MODEL)
    return out, jnp.stack(sls, 1), jnp.stack(aws, 1)


# speedup vs baseline: 58.6928x; 1.0617x over previous
import functools
import numpy as np
import jax
import jax.numpy as jnp
from jax import lax
from jax.experimental import pallas as pl
from jax.experimental.pallas import tpu as pltpu
from jax.experimental.pallas import tpu_sc as plsc

D_MODEL = 256
N_HEADS = 8
HEAD_DIM = 32
N_LEVELS = 4
N_POINTS = 4
D_FFN = 1024
SPATIAL = [(64, 64), (32, 32), (16, 16), (8, 8)]
B = 2
K_TOP = 1024
N_TOTAL = sum(h * w for h, w in SPATIAL)

NW = 32
QPW = (B * K_TOP) // NW
ROWS_PW = N_TOTAL // 16

def _col_consts():
    j = lax.broadcasted_iota(jnp.int32, (1, 128), 1)
    hj = jnp.right_shift(j, 4)
    lj = jnp.bitwise_and(jnp.right_shift(j, 2), 3)
    shift = 6 - lj
    wci = jnp.left_shift(jnp.int32(1), shift)
    wcf = wci.astype(jnp.float32)
    startc = jnp.where(lj == 0, 0, jnp.where(lj == 1, 4096,
                       jnp.where(lj == 2, 5120, 5376)))
    return hj, wci, wcf, startc


def _head_onehot():
    r = jnp.right_shift(lax.broadcasted_iota(jnp.int32, (128, 8), 0), 4)
    c = lax.broadcasted_iota(jnp.int32, (128, 8), 1)
    return (r == c).astype(jnp.float32)


def _value_body(src_ref, w1_ref, b1_ref, w2_ref, b2_ref, v1_ref, v2_ref):
    x = src_ref[...]
    v1_ref[...] = jnp.dot(x, w1_ref[...], preferred_element_type=jnp.float32) + b1_ref[...]
    v2_ref[...] = jnp.dot(x, w2_ref[...], preferred_element_type=jnp.float32) + b2_ref[...]


def _value_proj(src2d, w1, b1, w2, b2):
    blk = 680
    grid = (B * N_TOTAL) // blk
    return pl.pallas_call(
        _value_body,
        grid=(grid,),
        in_specs=[
            pl.BlockSpec((blk, D_MODEL), lambda i: (i, 0)),
            pl.BlockSpec((D_MODEL, D_MODEL), lambda i: (0, 0)),
            pl.BlockSpec((1, D_MODEL), lambda i: (0, 0)),
            pl.BlockSpec((D_MODEL, D_MODEL), lambda i: (0, 0)),
            pl.BlockSpec((1, D_MODEL), lambda i: (0, 0)),
        ],
        out_specs=[
            pl.BlockSpec((blk, D_MODEL), lambda i: (i, 0)),
            pl.BlockSpec((blk, D_MODEL), lambda i: (i, 0)),
        ],
        out_shape=[
            jax.ShapeDtypeStruct((B * N_TOTAL, D_MODEL), jnp.float32),
            jax.ShapeDtypeStruct((B * N_TOTAL, D_MODEL), jnp.float32),
        ],
    )(src2d, w1, b1.reshape(1, -1), w2, b2.reshape(1, -1))


def _qprep_body(tgt_ref, pos_ref, topk_ref, wox_ref, box_ref, woy_ref, boy_ref,
                wat_ref, bat_ref, slx_ref, sly_ref, att_ref, idx_ref, w_ref):
    blkq = tgt_ref.shape[0]
    pid = pl.program_id(0)
    b = (pid * blkq) // K_TOP
    query = tgt_ref[...] + pos_ref[...]

    offx = jnp.dot(query, wox_ref[...], preferred_element_type=jnp.float32) + box_ref[...]
    offy = jnp.dot(query, woy_ref[...], preferred_element_type=jnp.float32) + boy_ref[...]
    logits = jnp.dot(query, wat_ref[...], preferred_element_type=jnp.float32) + bat_ref[...]

    m = jnp.max(logits, axis=-1, keepdims=True)
    e = jnp.exp(logits - m)
    gh = _head_onehot()
    denom8 = jnp.dot(e, gh, preferred_element_type=jnp.float32)
    db = jnp.dot(denom8, gh.T, preferred_element_type=jnp.float32)
    att = e / db
    att_ref[...] = att

    n = topk_ref[...]
    lvl1 = (n >= 4096).astype(jnp.int32)
    lvl2 = (n >= 5120).astype(jnp.int32)
    lvl3 = (n >= 5376).astype(jnp.int32)
    start = jnp.where(n < 4096, 0, jnp.where(n < 5120, 4096,
                      jnp.where(n < 5376, 5120, 5376)))
    shift = 6 - (lvl1 + lvl2 + lvl3)
    wn = jnp.left_shift(jnp.int32(1), shift)
    rem = n - start
    row = jnp.right_shift(rem, shift)
    col = jnp.bitwise_and(rem, wn - 1)
    wn_f = wn.astype(jnp.float32)
    refx = (col.astype(jnp.float32) + 0.5) / wn_f
    refy = (row.astype(jnp.float32) + 0.5) / wn_f

    hj, wci, wcf, startc = _col_consts()
    hcf = wcf

    slx = refx + offx * (1.0 / wcf)
    sly = refy + offy * (1.0 / hcf)
    slx_ref[...] = slx
    sly_ref[...] = sly

    x = slx * wcf - 0.5
    y = sly * hcf - 0.5
    x0 = jnp.floor(x)
    y0 = jnp.floor(y)
    lwx = x - x0
    lwy = y - y0

    bn8 = b * (N_TOTAL * 8)
    corners = [
        (0.0, 0.0, (1 - lwx) * (1 - lwy)),
        (1.0, 0.0, lwx * (1 - lwy)),
        (0.0, 1.0, (1 - lwx) * lwy),
        (1.0, 1.0, lwx * lwy),
    ]
    for ci, (dx, dy, cw) in enumerate(corners):
        xi = x0 + dx
        yi = y0 + dy
        valid = ((xi >= 0) & (xi <= wcf - 1) & (yi >= 0) & (yi <= hcf - 1))
        xc = jnp.clip(xi, 0, wcf - 1).astype(jnp.int32)
        yc = jnp.clip(yi, 0, hcf - 1).astype(jnp.int32)
        loc = startc + yc * wci + xc
        idx = bn8 + loc * 8 + hj
        w = cw * att * valid.astype(jnp.float32)
        idx_ref[:, ci * 128:(ci + 1) * 128] = idx
        w_ref[:, ci * 128:(ci + 1) * 128] = w


def _qprep(tgt, pos_g, topk2d, wox, box, woy, boy, wat, bat):
    blkq = 256
    grid = (B * K_TOP) // blkq
    return pl.pallas_call(
        _qprep_body,
        grid=(grid,),
        in_specs=[
            pl.BlockSpec((blkq, D_MODEL), lambda i: (i, 0)),
            pl.BlockSpec((blkq, D_MODEL), lambda i: (i, 0)),
            pl.BlockSpec((blkq, 1), lambda i: (i, 0)),
            pl.BlockSpec((D_MODEL, 128), lambda i: (0, 0)),
            pl.BlockSpec((1, 128), lambda i: (0, 0)),
            pl.BlockSpec((D_MODEL, 128), lambda i: (0, 0)),
            pl.BlockSpec((1, 128), lambda i: (0, 0)),
            pl.BlockSpec((D_MODEL, 128), lambda i: (0, 0)),
            pl.BlockSpec((1, 128), lambda i: (0, 0)),
        ],
        out_specs=[
            pl.BlockSpec((blkq, 128), lambda i: (i, 0)),
            pl.BlockSpec((blkq, 128), lambda i: (i, 0)),
            pl.BlockSpec((blkq, 128), lambda i: (i, 0)),
            pl.BlockSpec((blkq, 512), lambda i: (i, 0)),
            pl.BlockSpec((blkq, 512), lambda i: (i, 0)),
        ],
        out_shape=[
            jax.ShapeDtypeStruct((B * K_TOP, 128), jnp.float32),
            jax.ShapeDtypeStruct((B * K_TOP, 128), jnp.float32),
            jax.ShapeDtypeStruct((B * K_TOP, 128), jnp.float32),
            jax.ShapeDtypeStruct((B * K_TOP, 512), jnp.int32),
            jax.ShapeDtypeStruct((B * K_TOP, 512), jnp.float32),
        ],
    )(tgt, pos_g, topk2d, wox, box.reshape(1, -1), woy, boy.reshape(1, -1),
      wat, bat.reshape(1, -1))


def _ln(x, g, b):
    m = jnp.mean(x, axis=-1, keepdims=True)
    xc = x - m
    v = jnp.mean(xc * xc, axis=-1, keepdims=True)
    return xc / jnp.sqrt(v + 1e-5) * g + b


def _post_body(attn_ref, tgt_ref, wo_ref, bo_ref, g1_ref, be1_ref,
               wf1_ref, bf1_ref, wf2_ref, bf2_ref, g2_ref, be2_ref,
               wvn_ref, bvn_ref, tgt_out_ref, vupd_ref):
    a = jnp.dot(attn_ref[...], wo_ref[...], preferred_element_type=jnp.float32) + bo_ref[...]
    t = _ln(tgt_ref[...] + a, g1_ref[...], be1_ref[...])
    ff1 = jnp.maximum(jnp.dot(t, wf1_ref[...], preferred_element_type=jnp.float32) + bf1_ref[...], 0.0)
    ff = jnp.dot(ff1, wf2_ref[...], preferred_element_type=jnp.float32) + bf2_ref[...]
    t2 = _ln(t + ff, g2_ref[...], be2_ref[...])
    tgt_out_ref[...] = t2
    vupd_ref[...] = jnp.dot(t2, wvn_ref[...], preferred_element_type=jnp.float32) + bvn_ref[...]


def _post(attn, tgt, wo, bo, g1, be1, wf1, bf1, wf2, bf2, g2, be2, wvn, bvn):
    blkq = 256
    grid = (B * K_TOP) // blkq
    full = lambda r, c: pl.BlockSpec((r, c), lambda i: (0, 0))
    return pl.pallas_call(
        _post_body,
        grid=(grid,),
        in_specs=[
            pl.BlockSpec((blkq, D_MODEL), lambda i: (i, 0)),
            pl.BlockSpec((blkq, D_MODEL), lambda i: (i, 0)),
            full(D_MODEL, D_MODEL), full(1, D_MODEL),
            full(1, D_MODEL), full(1, D_MODEL),
            full(D_MODEL, D_FFN), full(1, D_FFN),
            full(D_FFN, D_MODEL), full(1, D_MODEL),
            full(1, D_MODEL), full(1, D_MODEL),
            full(D_MODEL, D_MODEL), full(1, D_MODEL),
        ],
        out_specs=[
            pl.BlockSpec((blkq, D_MODEL), lambda i: (i, 0)),
            pl.BlockSpec((blkq, D_MODEL), lambda i: (i, 0)),
        ],
        out_shape=[
            jax.ShapeDtypeStruct((B * K_TOP, D_MODEL), jnp.float32),
            jax.ShapeDtypeStruct((B * K_TOP, D_MODEL), jnp.float32),
        ],
    )(attn, tgt, wo, bo.reshape(1, -1), g1.reshape(1, -1), be1.reshape(1, -1),
      wf1, bf1.reshape(1, -1), wf2, bf2.reshape(1, -1), g2.reshape(1, -1),
      be2.reshape(1, -1), wvn, bvn.reshape(1, -1))


def _sc_gather_body(src_hbm, pos_hbm, gidx_hbm, tgt_out, pos_out,
                    idx_v, rows_v, sem):
    c = lax.axis_index("c")
    s = lax.axis_index("s")
    wid = s * 2 + c
    base = wid * QPW
    pltpu.sync_copy(gidx_hbm.at[wid], idx_v)
    pltpu.async_copy(src_hbm.at[idx_v], rows_v, sem).wait()
    pltpu.sync_copy(rows_v, tgt_out.at[pl.ds(base, QPW)])
    pltpu.async_copy(pos_hbm.at[idx_v], rows_v, sem).wait()
    pltpu.sync_copy(rows_v, pos_out.at[pl.ds(base, QPW)])


def _sc_gather(src2d, pos2d, gidx):
    mesh = plsc.VectorSubcoreMesh(core_axis_name="c", subcore_axis_name="s")
    f = functools.partial(
        pl.kernel,
        out_type=[
            jax.ShapeDtypeStruct((B * K_TOP, D_MODEL), jnp.float32),
            jax.ShapeDtypeStruct((B * K_TOP, D_MODEL), jnp.float32),
        ],
        mesh=mesh,
        scratch_types=[
            pltpu.VMEM((QPW,), jnp.int32),
            pltpu.VMEM((QPW, D_MODEL), jnp.float32),
            pltpu.SemaphoreType.DMA,
        ],
    )(_sc_gather_body)
    return f(src2d, pos2d, gidx)


def _sc_attn_body(v_hbm, idx_hbm, w_hbm, out_hbm, idxw_v, w_v, rows0_v, rows1_v,
                  out_v, sem0, sem1):
    c = lax.axis_index("c")
    s = lax.axis_index("s")
    wid = s * 2 + c
    base = wid * QPW
    pltpu.sync_copy(idx_hbm.at[pl.ds(base * 512, QPW * 512)], idxw_v)
    pltpu.sync_copy(w_hbm.at[pl.ds(base * 512, QPW * 512)], w_v)

    def fire(q, rows_v, sem):
        for c4 in range(4):
            pltpu.async_copy(
                v_hbm.at[idxw_v.at[pl.ds(q * 512 + c4 * 128, 128)]],
                rows_v.at[pl.ds(c4 * 128, 128)], sem)

    def drain(rows_v, sem):
        for c4 in range(4):
            pltpu.make_async_copy(
                v_hbm.at[idxw_v.at[pl.ds(c4 * 128, 128)]],
                rows_v.at[pl.ds(c4 * 128, 128)], sem).wait()

    def compute(q, rows_v):
        for h in range(8):
            accs = []
            for c4 in range(4):
                acc0 = jnp.zeros((16,), jnp.float32)
                acc1 = jnp.zeros((16,), jnp.float32)
                w16 = w_v[pl.ds(q * 512 + c4 * 128 + h * 16, 16)]
                for k in range(16):
                    p = c4 * 128 + h * 16 + k
                    wb = w16[k]
                    acc0 = acc0 + wb * rows_v[p, pl.ds(0, 16)]
                    acc1 = acc1 + wb * rows_v[p, pl.ds(16, 16)]
                accs.append((acc0, acc1))
            out_v[q, pl.ds(h * 32, 16)] = (accs[0][0] + accs[1][0]) + (accs[2][0] + accs[3][0])
            out_v[q, pl.ds(h * 32 + 16, 16)] = (accs[0][1] + accs[1][1]) + (accs[2][1] + accs[3][1])

    fire(0, rows0_v, sem0)

    def body(q2, carry):
        q = q2 * 2
        fire(q + 1, rows1_v, sem1)
        drain(rows0_v, sem0)
        compute(q, rows0_v)
        fire(jnp.minimum(q + 2, QPW - 1), rows0_v, sem0)
        drain(rows1_v, sem1)
        compute(q + 1, rows1_v)
        return carry

    lax.fori_loop(0, QPW // 2, body, 0)
    drain(rows0_v, sem0)
    pltpu.sync_copy(out_v, out_hbm.at[pl.ds(base, QPW)])


def _sc_attn(v_rows, idx512, w512):
    mesh = plsc.VectorSubcoreMesh(core_axis_name="c", subcore_axis_name="s")
    f = functools.partial(
        pl.kernel,
        out_type=jax.ShapeDtypeStruct((B * K_TOP, D_MODEL), jnp.float32),
        mesh=mesh,
        scratch_types=[
            pltpu.VMEM((QPW * 512,), jnp.int32),
            pltpu.VMEM((QPW * 512,), jnp.float32),
            pltpu.VMEM((512, HEAD_DIM), jnp.float32),
            pltpu.VMEM((512, HEAD_DIM), jnp.float32),
            pltpu.VMEM((QPW, D_MODEL), jnp.float32),
            pltpu.SemaphoreType.DMA,
            pltpu.SemaphoreType.DMA,
        ],
        compiler_params=pltpu.CompilerParams(use_tc_tiling_on_sc=False),
    )(_sc_attn_body)
    return f(v_rows, idx512.reshape(-1), w512.reshape(-1))


def _sc_merge_body(base_hbm, rows_hbm, gidx_hbm, out_hbm, buf_v, idx_v, rows_v, sem):
    c = lax.axis_index("c")
    s = lax.axis_index("s")
    start = c * N_TOTAL + s * 336
    pltpu.sync_copy(base_hbm.at[pl.ds(start, 336)], buf_v)
    pltpu.sync_copy(buf_v, out_hbm.at[pl.ds(start, 336)])

    @pl.when(s == 0)
    def _():
        tail = c * N_TOTAL + 16 * 336
        pltpu.sync_copy(base_hbm.at[pl.ds(tail, 64)], buf_v.at[pl.ds(0, 64)])
        pltpu.sync_copy(buf_v.at[pl.ds(0, 64)], out_hbm.at[pl.ds(tail, 64)])

    plsc.subcore_barrier()
    chunk = c * 16 + s
    rbase = chunk * QPW
    pltpu.sync_copy(gidx_hbm.at[chunk], idx_v)
    pltpu.sync_copy(rows_hbm.at[pl.ds(rbase, QPW)], rows_v)
    pltpu.async_copy(rows_v, out_hbm.at[idx_v], sem).wait()


def _sc_merge(base2d, rows2d, gidx):
    mesh = plsc.VectorSubcoreMesh(core_axis_name="c", subcore_axis_name="s")
    f = functools.partial(
        pl.kernel,
        out_type=jax.ShapeDtypeStruct((B * N_TOTAL, D_MODEL), jnp.float32),
        mesh=mesh,
        scratch_types=[
            pltpu.VMEM((336, D_MODEL), jnp.float32),
            pltpu.VMEM((QPW,), jnp.int32),
            pltpu.VMEM((QPW, D_MODEL), jnp.float32),
            pltpu.SemaphoreType.DMA,
        ],
    )(_sc_merge_body)
    return f(base2d, rows2d, gidx)


def kernel(src, spatial_shapes, level_start_index, pos, topk_inds, params):
    src2d = src.reshape(B * N_TOTAL, D_MODEL)
    pos2d = pos.reshape(B * N_TOTAL, D_MODEL)
    gidx = (topk_inds + jnp.arange(B, dtype=jnp.int32)[:, None] * N_TOTAL)
    gidx = gidx.reshape(NW, QPW)
    topk2d = topk_inds.reshape(B * K_TOP, 1)

    p0, p1 = params
    v1, v2 = _value_proj(src2d, p0['W_val'], p0['b_val'], p1['W_val'], p1['b_val'])
    tgt, pos_g = _sc_gather(src2d, pos2d, gidx)

    sls = []
    aws = []
    v_cur = v1
    for li, p in enumerate(params):
        wox = p['W_off'][:, 0::2]
        box = p['b_off'][0::2]
        woy = p['W_off'][:, 1::2]
        boy = p['b_off'][1::2]
        slx, sly, att, idx512, w512 = _qprep(
            tgt, pos_g, topk2d, wox, box, woy, boy, p['W_att'], p['b_att'])
        sls.append(jnp.stack([slx, sly], -1).reshape(B, K_TOP, N_HEADS, N_LEVELS, N_POINTS, 2))
        aws.append(att.reshape(B, K_TOP, N_HEADS, N_LEVELS, N_POINTS))

        attn = _sc_attn(v_cur.reshape(B * N_TOTAL * N_HEADS, HEAD_DIM), idx512, w512)

        pn = params[min(li + 1, len(params) - 1)]
        tgt, vupd = _post(attn, tgt, p['W_out'], p['b_out'], p['g1'], p['be1'],
                          p['W_ff1'], p['b_ff1'], p['W_ff2'], p['b_ff2'],
                          p['g2'], p['be2'], pn['W_val'], pn['b_val'])
        if li == 0:
            v_cur = _sc_merge(v2, vupd, gidx)

    out = _sc_merge(src2d, tgt, gidx).reshape(B, N_TOTAL, D_MODEL)
    return out, jnp.stack(sls, 1), jnp.stack(aws, 1)
